# baseline (speedup 1.0000x reference)
import functools

import jax
import jax.numpy as jnp
from jax import lax
from jax.experimental import pallas as pl
from jax.experimental.pallas import tpu as pltpu
from jax.experimental.pallas import tpu_sc as plsc

NUM_CORES = 2
NUM_SUBCORES = 16
LANES = 16
GROUP = 256
NUM_WORKERS = NUM_CORES * NUM_SUBCORES
NPARAMS = 17

SC_GROUPS = 2560
TC_BLOCK_ROWS = 1024


def _vq_sc_body(chunk, n_groups, x_hbm, p_hbm, o_hbm, buf, par):
    c = lax.axis_index("c")
    s = lax.axis_index("s")
    wid = s * NUM_CORES + c
    base = wid * chunk

    pltpu.sync_copy(x_hbm.at[pl.ds(base, chunk)], buf)
    pltpu.sync_copy(p_hbm, par)

    a0, a1, a2, a3 = par[0], par[1], par[2], par[3]
    b0, b1, b2, b3 = par[4], par[5], par[6], par[7]
    da1, da2, da3 = par[8], par[9], par[10]
    db1, db2, db3 = par[11], par[12], par[13]
    dc1, dc2, dc3 = par[14], par[15], par[16]
    zero = jnp.zeros((LANES,), jnp.float32)

    def body(g, _):
        goff = g * GROUP
        for u in range(GROUP // (2 * LANES)):
            xo = goff + u * LANES
            yo = xo + (GROUP // 2)
            xv = buf[pl.ds(xo, LANES)]
            yv = buf[pl.ds(yo, LANES)]
            t1 = xv * da1 + yv * db1 + dc1
            t2 = xv * da2 + yv * db2 + dc2
            t3 = xv * da3 + yv * db3 + dc3
            g1 = t1 > zero
            m = jnp.maximum(t1, zero)
            ox = jnp.where(g1, a1, a0)
            oy = jnp.where(g1, b1, b0)
            g2 = t2 > m
            m = jnp.maximum(t2, m)
            ox = jnp.where(g2, a2, ox)
            oy = jnp.where(g2, b2, oy)
            g3 = t3 > m
            ox = jnp.where(g3, a3, ox)
            oy = jnp.where(g3, b3, oy)
            buf[pl.ds(xo, LANES)] = ox
            buf[pl.ds(yo, LANES)] = oy
        return 0

    lax.fori_loop(0, n_groups, body, 0)

    pltpu.sync_copy(buf, o_hbm.at[pl.ds(base, chunk)])


def _vq_tc_body(w_ref, x_ref, o_ref):
    a = x_ref[...]
    rows = a.shape[0]
    pd = pltpu.roll(a, rows - 1, 0)
    pu = pltpu.roll(a, 1, 0)
    even = (lax.broadcasted_iota(jnp.int32, a.shape, 0) & 1) == 0
    xv = jnp.where(even, a, pu)
    yv = jnp.where(even, pd, a)
    a0, a1, a2, a3 = w_ref[0, 0], w_ref[0, 1], w_ref[0, 2], w_ref[0, 3]
    b0, b1, b2, b3 = w_ref[1, 0], w_ref[1, 1], w_ref[1, 2], w_ref[1, 3]
    c0 = -0.5 * (a0 * a0 + b0 * b0)
    c1 = -0.5 * (a1 * a1 + b1 * b1)
    c2 = -0.5 * (a2 * a2 + b2 * b2)
    c3 = -0.5 * (a3 * a3 + b3 * b3)
    t1 = xv * (a1 - a0) + yv * (b1 - b0) + (c1 - c0)
    t2 = xv * (a2 - a0) + yv * (b2 - b0) + (c2 - c0)
    t3 = xv * (a3 - a0) + yv * (b3 - b0) + (c3 - c0)
    g1 = t1 > 0.0
    m = jnp.maximum(t1, 0.0)
    ox = jnp.where(g1, a1, a0)
    oy = jnp.where(g1, b1, b0)
    g2 = t2 > m
    m = jnp.maximum(t2, m)
    ox = jnp.where(g2, a2, ox)
    oy = jnp.where(g2, b2, oy)
    g3 = t3 > m
    ox = jnp.where(g3, a3, ox)
    oy = jnp.where(g3, b3, oy)
    o_ref[...] = jnp.where(even, ox, oy)


def kernel(x, w, Centroid_sum, Centroid_n):
    n, d = x.shape
    total = n * d
    n_rows = total // 128

    xt = jnp.transpose(jnp.reshape(x, (n // 128, 128, d)), (0, 2, 1))
    xflat = jnp.reshape(xt, (total,))

    a = w[0]
    b = w[1]
    c = -0.5 * jnp.sum(w * w, axis=0)
    scal = jnp.concatenate([
        a, b, a[1:] - a[0], b[1:] - b[0], c[1:] - c[0]
    ]).astype(jnp.float32)
    params = jnp.broadcast_to(scal[:, None], (NPARAMS, LANES))

    sc_words = SC_GROUPS * GROUP
    sc_rows = sc_words // 128
    chunk = sc_words // NUM_WORKERS
    n_groups = chunk // GROUP

    mesh = plsc.VectorSubcoreMesh(
        core_axis_name="c", subcore_axis_name="s",
        num_cores=NUM_CORES, num_subcores=NUM_SUBCORES,
    )
    sc_run = pl.kernel(
        functools.partial(_vq_sc_body, chunk, n_groups),
        out_type=jax.ShapeDtypeStruct((sc_words,), jnp.float32),
        mesh=mesh,
        scratch_types=[
            pltpu.VMEM((chunk,), jnp.float32),
            pltpu.VMEM((NPARAMS, LANES), jnp.float32),
        ],
        compiler_params=pltpu.CompilerParams(needs_layout_passes=False),
    )
    sc_out = sc_run(xflat, params)

    tc_rows = n_rows - sc_rows
    x2d = jnp.reshape(xflat, (n_rows, 128))
    tc_out = pl.pallas_call(
        _vq_tc_body,
        grid=(tc_rows // TC_BLOCK_ROWS,),
        in_specs=[
            pl.BlockSpec(memory_space=pltpu.SMEM),
            pl.BlockSpec(
                (TC_BLOCK_ROWS, 128),
                lambda i: (sc_rows // TC_BLOCK_ROWS + i, 0),
            ),
        ],
        out_specs=pl.BlockSpec(
            (TC_BLOCK_ROWS, 128),
            lambda i: (sc_rows // TC_BLOCK_ROWS + i, 0),
        ),
        out_shape=jax.ShapeDtypeStruct((n_rows, 128), jnp.float32),
    )(w, x2d)

    out2d = lax.dynamic_update_slice(
        tc_out, jnp.reshape(sc_out, (sc_rows, 128)), (0, 0))

    out3 = jnp.reshape(out2d, (n // 128, d, 128))
    return jnp.reshape(jnp.transpose(out3, (0, 2, 1)), (n, d))


# speedup vs baseline: 1.1300x; 1.1300x over previous
import functools

import jax
import jax.numpy as jnp
from jax import lax
from jax.experimental import pallas as pl
from jax.experimental.pallas import tpu as pltpu
from jax.experimental.pallas import tpu_sc as plsc

NUM_CORES = 2
NUM_SUBCORES = 16
LANES = 16
GROUP = 256
NUM_WORKERS = NUM_CORES * NUM_SUBCORES
NPARAMS = 17

SC_GROUPS = 2560
TC_BLOCK_ROWS = 1024


def _vq_sc_body(chunk, n_groups, x_hbm, p_hbm, o_hbm, buf, par):
    c = lax.axis_index("c")
    s = lax.axis_index("s")
    wid = s * NUM_CORES + c
    base = wid * chunk

    pltpu.sync_copy(x_hbm.at[pl.ds(base, chunk)], buf)
    pltpu.sync_copy(p_hbm, par)

    wrow0 = par[0]
    wrow1 = par[1]

    def lane(v, j):
        idx = jnp.full((LANES,), j, jnp.int32)
        return v.at[idx].get(mode="promise_in_bounds")

    a0, a1, a2, a3 = (lane(wrow0, j) for j in range(4))
    b0, b1, b2, b3 = (lane(wrow1, j) for j in range(4))
    c0 = -0.5 * (a0 * a0 + b0 * b0)
    da1, da2, da3 = a1 - a0, a2 - a0, a3 - a0
    db1, db2, db3 = b1 - b0, b2 - b0, b3 - b0
    dc1 = -0.5 * (a1 * a1 + b1 * b1) - c0
    dc2 = -0.5 * (a2 * a2 + b2 * b2) - c0
    dc3 = -0.5 * (a3 * a3 + b3 * b3) - c0
    zero = jnp.zeros((LANES,), jnp.float32)

    def body(g, _):
        goff = g * GROUP
        for u in range(GROUP // (2 * LANES)):
            xo = goff + u * LANES
            yo = xo + (GROUP // 2)
            xv = buf[pl.ds(xo, LANES)]
            yv = buf[pl.ds(yo, LANES)]
            t1 = xv * da1 + yv * db1 + dc1
            t2 = xv * da2 + yv * db2 + dc2
            t3 = xv * da3 + yv * db3 + dc3
            g1 = t1 > zero
            m = jnp.maximum(t1, zero)
            ox = jnp.where(g1, a1, a0)
            oy = jnp.where(g1, b1, b0)
            g2 = t2 > m
            m = jnp.maximum(t2, m)
            ox = jnp.where(g2, a2, ox)
            oy = jnp.where(g2, b2, oy)
            g3 = t3 > m
            ox = jnp.where(g3, a3, ox)
            oy = jnp.where(g3, b3, oy)
            buf[pl.ds(xo, LANES)] = ox
            buf[pl.ds(yo, LANES)] = oy
        return 0

    lax.fori_loop(0, n_groups, body, 0)

    pltpu.sync_copy(buf, o_hbm.at[pl.ds(base, chunk)])


def _vq_tc_body(w_ref, x_ref, o_ref):
    a = x_ref[...]
    rows = a.shape[0]
    pd = pltpu.roll(a, rows - 1, 0)
    pu = pltpu.roll(a, 1, 0)
    even = (lax.broadcasted_iota(jnp.int32, a.shape, 0) & 1) == 0
    xv = jnp.where(even, a, pu)
    yv = jnp.where(even, pd, a)
    a0, a1, a2, a3 = w_ref[0, 0], w_ref[0, 1], w_ref[0, 2], w_ref[0, 3]
    b0, b1, b2, b3 = w_ref[1, 0], w_ref[1, 1], w_ref[1, 2], w_ref[1, 3]
    c0 = -0.5 * (a0 * a0 + b0 * b0)
    c1 = -0.5 * (a1 * a1 + b1 * b1)
    c2 = -0.5 * (a2 * a2 + b2 * b2)
    c3 = -0.5 * (a3 * a3 + b3 * b3)
    t1 = xv * (a1 - a0) + yv * (b1 - b0) + (c1 - c0)
    t2 = xv * (a2 - a0) + yv * (b2 - b0) + (c2 - c0)
    t3 = xv * (a3 - a0) + yv * (b3 - b0) + (c3 - c0)
    g1 = t1 > 0.0
    m = jnp.maximum(t1, 0.0)
    ox = jnp.where(g1, a1, a0)
    oy = jnp.where(g1, b1, b0)
    g2 = t2 > m
    m = jnp.maximum(t2, m)
    ox = jnp.where(g2, a2, ox)
    oy = jnp.where(g2, b2, oy)
    g3 = t3 > m
    ox = jnp.where(g3, a3, ox)
    oy = jnp.where(g3, b3, oy)
    o_ref[...] = jnp.where(even, ox, oy)


def kernel(x, w, Centroid_sum, Centroid_n):
    n, d = x.shape
    total = n * d
    n_rows = total // 128

    xt = jnp.transpose(jnp.reshape(x, (n // 128, 128, d)), (0, 2, 1))
    xflat = jnp.reshape(xt, (total,))

    w_pad = lax.pad(w.astype(jnp.float32), jnp.float32(0),
                    [(0, 0, 0), (0, LANES - 4, 0)])

    sc_words = SC_GROUPS * GROUP
    sc_rows = sc_words // 128
    chunk = sc_words // NUM_WORKERS
    n_groups = chunk // GROUP

    mesh = plsc.VectorSubcoreMesh(
        core_axis_name="c", subcore_axis_name="s",
        num_cores=NUM_CORES, num_subcores=NUM_SUBCORES,
    )
    sc_run = pl.kernel(
        functools.partial(_vq_sc_body, chunk, n_groups),
        out_type=jax.ShapeDtypeStruct((sc_words,), jnp.float32),
        mesh=mesh,
        scratch_types=[
            pltpu.VMEM((chunk,), jnp.float32),
            pltpu.VMEM((2, LANES), jnp.float32),
        ],
        compiler_params=pltpu.CompilerParams(needs_layout_passes=False),
    )
    sc_out = sc_run(xflat, w_pad)

    tc_rows = n_rows - sc_rows
    x2d = jnp.reshape(xflat, (n_rows, 128))
    tc_out = pl.pallas_call(
        _vq_tc_body,
        grid=(tc_rows // TC_BLOCK_ROWS,),
        in_specs=[
            pl.BlockSpec(memory_space=pltpu.SMEM),
            pl.BlockSpec(
                (TC_BLOCK_ROWS, 128),
                lambda i: (sc_rows // TC_BLOCK_ROWS + i, 0),
            ),
        ],
        out_specs=pl.BlockSpec(
            (TC_BLOCK_ROWS, 128),
            lambda i: (sc_rows // TC_BLOCK_ROWS + i, 0),
        ),
        out_shape=jax.ShapeDtypeStruct((n_rows, 128), jnp.float32),
    )(w, x2d)

    out2d = lax.dynamic_update_slice(
        tc_out, jnp.reshape(sc_out, (sc_rows, 128)), (0, 0))

    out3 = jnp.reshape(out2d, (n // 128, d, 128))
    return jnp.reshape(jnp.transpose(out3, (0, 2, 1)), (n, d))


# speedup vs baseline: 1.1342x; 1.0037x over previous
import functools

import jax
import jax.numpy as jnp
from jax import lax
from jax.experimental import pallas as pl
from jax.experimental.pallas import tpu as pltpu
from jax.experimental.pallas import tpu_sc as plsc

NUM_CORES = 2
NUM_SUBCORES = 16
LANES = 16
GROUP = 256
NUM_WORKERS = NUM_CORES * NUM_SUBCORES
NPARAMS = 17

SC_GROUPS = 2560
TC_BLOCK_ROWS = 1024


def _vq_sc_body(chunk, n_groups, x_hbm, p_hbm, o_hbm, buf, par):
    c = lax.axis_index("c")
    s = lax.axis_index("s")
    wid = s * NUM_CORES + c
    base = wid * chunk

    pltpu.sync_copy(x_hbm.at[pl.ds(base, chunk)], buf)
    pltpu.sync_copy(p_hbm, par)

    wrow0 = par[0]
    wrow1 = par[1]

    def lane(v, j):
        idx = jnp.full((LANES,), j, jnp.int32)
        return v.at[idx].get(mode="promise_in_bounds")

    a0, a1, a2, a3 = (lane(wrow0, j) for j in range(4))
    b0, b1, b2, b3 = (lane(wrow1, j) for j in range(4))
    c0 = -0.5 * (a0 * a0 + b0 * b0)
    da1, da2, da3 = a1 - a0, a2 - a0, a3 - a0
    db1, db2, db3 = b1 - b0, b2 - b0, b3 - b0
    dc1 = -0.5 * (a1 * a1 + b1 * b1) - c0
    dc2 = -0.5 * (a2 * a2 + b2 * b2) - c0
    dc3 = -0.5 * (a3 * a3 + b3 * b3) - c0
    zero = jnp.zeros((LANES,), jnp.float32)

    def body(g, _):
        goff = g * GROUP
        for u in range(GROUP // (2 * LANES)):
            xo = goff + u * LANES
            yo = xo + (GROUP // 2)
            xv = buf[pl.ds(xo, LANES)]
            yv = buf[pl.ds(yo, LANES)]
            t1 = xv * da1 + yv * db1 + dc1
            t2 = xv * da2 + yv * db2 + dc2
            t3 = xv * da3 + yv * db3 + dc3
            g1 = t1 > zero
            m = jnp.maximum(t1, zero)
            ox = jnp.where(g1, a1, a0)
            oy = jnp.where(g1, b1, b0)
            g2 = t2 > m
            m = jnp.maximum(t2, m)
            ox = jnp.where(g2, a2, ox)
            oy = jnp.where(g2, b2, oy)
            g3 = t3 > m
            ox = jnp.where(g3, a3, ox)
            oy = jnp.where(g3, b3, oy)
            buf[pl.ds(xo, LANES)] = ox
            buf[pl.ds(yo, LANES)] = oy
        return 0

    lax.fori_loop(0, n_groups, body, 0)

    pltpu.sync_copy(buf, o_hbm.at[pl.ds(base, chunk)])


def _vq_tc_body(w_ref, x_ref, o_ref):
    a = x_ref[...]
    rows = a.shape[0]
    pd = pltpu.roll(a, rows - 1, 0)
    pu = pltpu.roll(a, 1, 0)
    even = (lax.broadcasted_iota(jnp.int32, a.shape, 0) & 1) == 0
    xv = jnp.where(even, a, pu)
    yv = jnp.where(even, pd, a)
    a0, a1, a2, a3 = w_ref[0, 0], w_ref[0, 1], w_ref[0, 2], w_ref[0, 3]
    b0, b1, b2, b3 = w_ref[1, 0], w_ref[1, 1], w_ref[1, 2], w_ref[1, 3]
    c0 = -0.5 * (a0 * a0 + b0 * b0)
    c1 = -0.5 * (a1 * a1 + b1 * b1)
    c2 = -0.5 * (a2 * a2 + b2 * b2)
    c3 = -0.5 * (a3 * a3 + b3 * b3)
    t1 = xv * (a1 - a0) + yv * (b1 - b0) + (c1 - c0)
    t2 = xv * (a2 - a0) + yv * (b2 - b0) + (c2 - c0)
    t3 = xv * (a3 - a0) + yv * (b3 - b0) + (c3 - c0)
    g1 = t1 > 0.0
    m = jnp.maximum(t1, 0.0)
    ox = jnp.where(g1, a1, a0)
    oy = jnp.where(g1, b1, b0)
    g2 = t2 > m
    m = jnp.maximum(t2, m)
    ox = jnp.where(g2, a2, ox)
    oy = jnp.where(g2, b2, oy)
    g3 = t3 > m
    ox = jnp.where(g3, a3, ox)
    oy = jnp.where(g3, b3, oy)
    o_ref[...] = jnp.where(even, ox, oy)


def kernel(x, w, Centroid_sum, Centroid_n):
    n, d = x.shape
    total = n * d
    n_rows = total // 128

    xt = jnp.transpose(jnp.reshape(x, (n // 128, 128, d)), (0, 2, 1))
    xflat = jnp.reshape(xt, (total,))

    w_pad = lax.pad(w.astype(jnp.float32), jnp.float32(0),
                    [(0, 0, 0), (0, LANES - 4, 0)])

    sc_words = SC_GROUPS * GROUP
    sc_rows = sc_words // 128
    chunk = sc_words // NUM_WORKERS
    n_groups = chunk // GROUP

    mesh = plsc.VectorSubcoreMesh(
        core_axis_name="c", subcore_axis_name="s",
        num_cores=NUM_CORES, num_subcores=NUM_SUBCORES,
    )
    sc_run = pl.kernel(
        functools.partial(_vq_sc_body, chunk, n_groups),
        out_type=jax.ShapeDtypeStruct((sc_words,), jnp.float32),
        mesh=mesh,
        scratch_types=[
            pltpu.VMEM((chunk,), jnp.float32),
            pltpu.VMEM((2, LANES), jnp.float32),
        ],
        compiler_params=pltpu.CompilerParams(
            needs_layout_passes=False, skip_device_barrier=True),
    )
    sc_out = sc_run(xflat, w_pad)

    tc_rows = n_rows - sc_rows
    x2d = jnp.reshape(xflat, (n_rows, 128))
    tc_out = pl.pallas_call(
        _vq_tc_body,
        grid=(tc_rows // TC_BLOCK_ROWS,),
        in_specs=[
            pl.BlockSpec(memory_space=pltpu.SMEM),
            pl.BlockSpec(
                (TC_BLOCK_ROWS, 128),
                lambda i: (sc_rows // TC_BLOCK_ROWS + i, 0),
            ),
        ],
        out_specs=pl.BlockSpec(
            (TC_BLOCK_ROWS, 128),
            lambda i: (sc_rows // TC_BLOCK_ROWS + i, 0),
        ),
        out_shape=jax.ShapeDtypeStruct((n_rows, 128), jnp.float32),
    )(w, x2d)

    out2d = lax.dynamic_update_slice(
        tc_out, jnp.reshape(sc_out, (sc_rows, 128)), (0, 0))

    out3 = jnp.reshape(out2d, (n // 128, d, 128))
    return jnp.reshape(jnp.transpose(out3, (0, 2, 1)), (n, d))


# speedup vs baseline: 1.1349x; 1.0006x over previous
import functools

import jax
import jax.numpy as jnp
from jax import lax
from jax.experimental import pallas as pl
from jax.experimental.pallas import tpu as pltpu
from jax.experimental.pallas import tpu_sc as plsc

NUM_CORES = 2
NUM_SUBCORES = 16
LANES = 16
GROUP = 256
NUM_WORKERS = NUM_CORES * NUM_SUBCORES
NPARAMS = 17

SC_GROUPS = 3072
TC_BLOCK_ROWS = 2048


def _vq_sc_body(chunk, n_groups, x_hbm, p_hbm, o_hbm, buf, par):
    c = lax.axis_index("c")
    s = lax.axis_index("s")
    wid = s * NUM_CORES + c
    base = wid * chunk

    pltpu.sync_copy(x_hbm.at[pl.ds(base, chunk)], buf)
    pltpu.sync_copy(p_hbm, par)

    wrow0 = par[0]
    wrow1 = par[1]

    def lane(v, j):
        idx = jnp.full((LANES,), j, jnp.int32)
        return v.at[idx].get(mode="promise_in_bounds")

    a0, a1, a2, a3 = (lane(wrow0, j) for j in range(4))
    b0, b1, b2, b3 = (lane(wrow1, j) for j in range(4))
    c0 = -0.5 * (a0 * a0 + b0 * b0)
    da1, da2, da3 = a1 - a0, a2 - a0, a3 - a0
    db1, db2, db3 = b1 - b0, b2 - b0, b3 - b0
    dc1 = -0.5 * (a1 * a1 + b1 * b1) - c0
    dc2 = -0.5 * (a2 * a2 + b2 * b2) - c0
    dc3 = -0.5 * (a3 * a3 + b3 * b3) - c0
    zero = jnp.zeros((LANES,), jnp.float32)

    def body(g, _):
        goff = g * GROUP
        for u in range(GROUP // (2 * LANES)):
            xo = goff + u * LANES
            yo = xo + (GROUP // 2)
            xv = buf[pl.ds(xo, LANES)]
            yv = buf[pl.ds(yo, LANES)]
            t1 = xv * da1 + yv * db1 + dc1
            t2 = xv * da2 + yv * db2 + dc2
            t3 = xv * da3 + yv * db3 + dc3
            g1 = t1 > zero
            m = jnp.maximum(t1, zero)
            ox = jnp.where(g1, a1, a0)
            oy = jnp.where(g1, b1, b0)
            g2 = t2 > m
            m = jnp.maximum(t2, m)
            ox = jnp.where(g2, a2, ox)
            oy = jnp.where(g2, b2, oy)
            g3 = t3 > m
            ox = jnp.where(g3, a3, ox)
            oy = jnp.where(g3, b3, oy)
            buf[pl.ds(xo, LANES)] = ox
            buf[pl.ds(yo, LANES)] = oy
        return 0

    lax.fori_loop(0, n_groups, body, 0)

    pltpu.sync_copy(buf, o_hbm.at[pl.ds(base, chunk)])


def _vq_tc_body(w_ref, x_ref, o_ref):
    a = x_ref[...]
    rows = a.shape[0]
    pd = pltpu.roll(a, rows - 1, 0)
    pu = pltpu.roll(a, 1, 0)
    even = (lax.broadcasted_iota(jnp.int32, a.shape, 0) & 1) == 0
    xv = jnp.where(even, a, pu)
    yv = jnp.where(even, pd, a)
    a0, a1, a2, a3 = w_ref[0, 0], w_ref[0, 1], w_ref[0, 2], w_ref[0, 3]
    b0, b1, b2, b3 = w_ref[1, 0], w_ref[1, 1], w_ref[1, 2], w_ref[1, 3]
    c0 = -0.5 * (a0 * a0 + b0 * b0)
    c1 = -0.5 * (a1 * a1 + b1 * b1)
    c2 = -0.5 * (a2 * a2 + b2 * b2)
    c3 = -0.5 * (a3 * a3 + b3 * b3)
    t1 = xv * (a1 - a0) + yv * (b1 - b0) + (c1 - c0)
    t2 = xv * (a2 - a0) + yv * (b2 - b0) + (c2 - c0)
    t3 = xv * (a3 - a0) + yv * (b3 - b0) + (c3 - c0)
    g1 = t1 > 0.0
    m = jnp.maximum(t1, 0.0)
    ox = jnp.where(g1, a1, a0)
    oy = jnp.where(g1, b1, b0)
    g2 = t2 > m
    m = jnp.maximum(t2, m)
    ox = jnp.where(g2, a2, ox)
    oy = jnp.where(g2, b2, oy)
    g3 = t3 > m
    ox = jnp.where(g3, a3, ox)
    oy = jnp.where(g3, b3, oy)
    o_ref[...] = jnp.where(even, ox, oy)


def kernel(x, w, Centroid_sum, Centroid_n):
    n, d = x.shape
    total = n * d
    n_rows = total // 128

    xt = jnp.transpose(jnp.reshape(x, (n // 128, 128, d)), (0, 2, 1))
    xflat = jnp.reshape(xt, (total,))

    w_pad = lax.pad(w.astype(jnp.float32), jnp.float32(0),
                    [(0, 0, 0), (0, LANES - 4, 0)])

    sc_words = SC_GROUPS * GROUP
    sc_rows = sc_words // 128
    chunk = sc_words // NUM_WORKERS
    n_groups = chunk // GROUP

    mesh = plsc.VectorSubcoreMesh(
        core_axis_name="c", subcore_axis_name="s",
        num_cores=NUM_CORES, num_subcores=NUM_SUBCORES,
    )
    sc_run = pl.kernel(
        functools.partial(_vq_sc_body, chunk, n_groups),
        out_type=jax.ShapeDtypeStruct((sc_words,), jnp.float32),
        mesh=mesh,
        scratch_types=[
            pltpu.VMEM((chunk,), jnp.float32),
            pltpu.VMEM((2, LANES), jnp.float32),
        ],
        compiler_params=pltpu.CompilerParams(needs_layout_passes=False),
    )
    sc_out = sc_run(xflat, w_pad)

    tc_rows = n_rows - sc_rows
    x2d = jnp.reshape(xflat, (n_rows, 128))
    tc_out = pl.pallas_call(
        _vq_tc_body,
        grid=(tc_rows // TC_BLOCK_ROWS,),
        in_specs=[
            pl.BlockSpec(memory_space=pltpu.SMEM),
            pl.BlockSpec(
                (TC_BLOCK_ROWS, 128),
                lambda i: (sc_rows // TC_BLOCK_ROWS + i, 0),
            ),
        ],
        out_specs=pl.BlockSpec(
            (TC_BLOCK_ROWS, 128),
            lambda i: (sc_rows // TC_BLOCK_ROWS + i, 0),
        ),
        out_shape=jax.ShapeDtypeStruct((n_rows, 128), jnp.float32),
    )(w, x2d)

    out2d = lax.dynamic_update_slice(
        tc_out, jnp.reshape(sc_out, (sc_rows, 128)), (0, 0))

    out3 = jnp.reshape(out2d, (n // 128, d, 128))
    return jnp.reshape(jnp.transpose(out3, (0, 2, 1)), (n, d))
